# Initial kernel scaffold; baseline (speedup 1.0000x reference)
import jax
import jax.numpy as jnp
from jax.experimental import pallas as pl


def kernel(images, targets):
    raise NotImplementedError("write your pallas kernel here")


# speedup vs baseline: 1.0047x; 1.0047x over previous
import numpy as np
import jax
import jax.numpy as jnp
from jax.experimental import pallas as pl

_IN = 1024
_OUT = 640
_MEAN = np.array([0.485, 0.456, 0.406], np.float64)
_STD = np.array([0.229, 0.224, 0.225], np.float64)


def _weight_mat(in_size, out_size):
    scale = out_size / in_size
    kernel_scale = 1.0 / scale
    sample_f = (np.arange(out_size, dtype=np.float64) + 0.5) / scale - 0.5
    x = np.abs(sample_f[None, :] - np.arange(in_size, dtype=np.float64)[:, None])
    w = np.maximum(0.0, 1.0 - x / kernel_scale)
    return w / w.sum(axis=0, keepdims=True)


_W = _weight_mat(_IN, _OUT)
_WHT = jnp.asarray(_W.T, jnp.float32)
_WW = jnp.asarray(_W, jnp.float32)

_INV_STD = tuple(np.float32(1.0 / s) for s in _STD)
_BIAS = tuple(np.float32(-m / s) for m, s in zip(_MEAN, _STD))


def _body(x_ref, wht_ref, ww_ref, o_ref):
    c = jax.lax.rem(pl.program_id(0), 3)
    inv = jnp.where(c == 0, _INV_STD[0], jnp.where(c == 1, _INV_STD[1], _INV_STD[2]))
    bias = jnp.where(c == 0, _BIAS[0], jnp.where(c == 1, _BIAS[1], _BIAS[2]))
    x = x_ref[0]
    y = jax.lax.dot(wht_ref[...], x,
                    precision=jax.lax.Precision.HIGHEST,
                    preferred_element_type=jnp.float32)
    z = jax.lax.dot(y, ww_ref[...],
                    precision=jax.lax.Precision.HIGHEST,
                    preferred_element_type=jnp.float32)
    o_ref[0] = z * inv + bias


def kernel(images, targets):
    B, C, H, W = images.shape
    x = images.reshape(B * C, H, W)
    out = pl.pallas_call(
        _body,
        grid=(B * C,),
        in_specs=[
            pl.BlockSpec((1, H, W), lambda i: (i, 0, 0)),
            pl.BlockSpec((_OUT, _IN), lambda i: (0, 0)),
            pl.BlockSpec((_IN, _OUT), lambda i: (0, 0)),
        ],
        out_specs=pl.BlockSpec((1, _OUT, _OUT), lambda i: (i, 0, 0)),
        out_shape=jax.ShapeDtypeStruct((B * C, _OUT, _OUT), jnp.float32),
    )(x, _WHT, _WW)
    return out.reshape(B, C, _OUT, _OUT)


# speedup vs baseline: 1.1454x; 1.1400x over previous
import numpy as np
import jax
import jax.numpy as jnp
from jax.experimental import pallas as pl

_IN = 1024
_OUT = 640
_MEAN = np.array([0.485, 0.456, 0.406], np.float64)
_STD = np.array([0.229, 0.224, 0.225], np.float64)


def _weight_mat(in_size, out_size):
    scale = out_size / in_size
    kernel_scale = 1.0 / scale
    sample_f = (np.arange(out_size, dtype=np.float64) + 0.5) / scale - 0.5
    x = np.abs(sample_f[None, :] - np.arange(in_size, dtype=np.float64)[:, None])
    w = np.maximum(0.0, 1.0 - x / kernel_scale)
    return w / w.sum(axis=0, keepdims=True)


_W = _weight_mat(_IN, _OUT)
_WW = jnp.asarray(_W, jnp.float32)

_INV_STD = tuple(np.float32(1.0 / s) for s in _STD)
_BIAS = tuple(np.float32(-m / s) for m, s in zip(_MEAN, _STD))

_PHASE = []
for _p in range(5):
    _col = _W[:, 5 + _p]
    _nz = np.nonzero(_col > 0)[0]
    _PHASE.append(((_nz - 8).tolist(), _col[_nz].tolist()))
_EDGE0 = _W[0, 0] / _PHASE[0][1][1]
_EDGE1 = _W[_IN - 2, _OUT - 1] / _PHASE[4][1][0]
assert np.allclose(_W[1, 0], _PHASE[0][1][2] * _EDGE0)
assert np.allclose(_W[_IN - 1, _OUT - 1], _PHASE[4][1][1] * _EDGE1)


def _vertical_pass(x):
    n = x.shape[1]
    x3 = x.reshape(128, 8, n)
    rows = [x3[:, c, :] for c in range(8)]
    zrow = jnp.zeros((1, n), jnp.float32)
    prev7 = jnp.concatenate([zrow, rows[7][:-1]], axis=0)
    next0 = jnp.concatenate([rows[0][1:], zrow], axis=0)

    def src(o):
        return prev7 if o == -1 else (next0 if o == 8 else rows[o])

    ys = []
    for offs, ws in _PHASE:
        acc = np.float32(ws[0]) * src(offs[0])
        for o, w in zip(offs[1:], ws[1:]):
            acc = acc + np.float32(w) * src(o)
        ys.append(acc)
    g = jax.lax.broadcasted_iota(jnp.int32, (128, n), 0)
    ys[0] = jnp.where(g == 0, ys[0] * np.float32(_EDGE0), ys[0])
    ys[4] = jnp.where(g == 127, ys[4] * np.float32(_EDGE1), ys[4])
    return jnp.stack(ys, axis=1).reshape(_OUT, n)


def _body(x_ref, ww_ref, o_ref):
    c = jax.lax.rem(pl.program_id(0), 3)
    inv = jnp.where(c == 0, _INV_STD[0], jnp.where(c == 1, _INV_STD[1], _INV_STD[2]))
    bias = jnp.where(c == 0, _BIAS[0], jnp.where(c == 1, _BIAS[1], _BIAS[2]))
    y = _vertical_pass(x_ref[0])
    z = jax.lax.dot(y, ww_ref[...],
                    precision=jax.lax.Precision.HIGHEST,
                    preferred_element_type=jnp.float32)
    o_ref[0] = z * inv + bias


def kernel(images, targets):
    B, C, H, W = images.shape
    x = images.reshape(B * C, H, W)
    out = pl.pallas_call(
        _body,
        grid=(B * C,),
        in_specs=[
            pl.BlockSpec((1, H, W), lambda i: (i, 0, 0)),
            pl.BlockSpec((_IN, _OUT), lambda i: (0, 0)),
        ],
        out_specs=pl.BlockSpec((1, _OUT, _OUT), lambda i: (i, 0, 0)),
        out_shape=jax.ShapeDtypeStruct((B * C, _OUT, _OUT), jnp.float32),
    )(x, _WW)
    return out.reshape(B, C, _OUT, _OUT)
